# baseline (speedup 1.0000x reference)
import jax
import jax.numpy as jnp
from jax.experimental import pallas as pl

N_V = 10000; N_A = 10000; D = 128; H = 128; L = 4; OUT = 128; G = 32


def _gcn(x, ei, ew, W, b, n_dst):
    h = x @ W
    msg = h[ei[0]] * ew[:, None]
    return jax.ops.segment_sum(msg, ei[1], num_segments=n_dst) + b


def _gat(x_s, x_d, ei, ew, Ws, Wd, a_s, a_d, b, n_dst):
    hs = x_s @ Ws
    hd = x_d @ Wd
    e = jax.nn.leaky_relu(hs[ei[0]] @ a_s + hd[ei[1]] @ a_d, 0.2)
    m = jax.ops.segment_max(e, ei[1], num_segments=n_dst)
    ex = jnp.exp(e - m[ei[1]])
    s = jax.ops.segment_sum(ex, ei[1], num_segments=n_dst)
    alpha = ex / (s[ei[1]] + 1e-16)
    msg = hs[ei[0]] * (alpha * ew)[:, None]
    return jax.ops.segment_sum(msg, ei[1], num_segments=n_dst) + b


def _graph_layernorm(x, batch, gamma, beta):
    d = x.shape[1]
    cnt = jax.ops.segment_sum(jnp.ones((x.shape[0],), x.dtype), batch, num_segments=G)
    denom = jnp.maximum(cnt * d, 1.0)
    mean = jax.ops.segment_sum(x.sum(axis=1), batch, num_segments=G) / denom
    xc = x - mean[batch][:, None]
    var = jax.ops.segment_sum((xc * xc).sum(axis=1), batch, num_segments=G) / denom
    return xc / jnp.sqrt(var[batch][:, None] + 1e-5) * gamma + beta


def _final_mm_kernel(ge_ref, w_ref, b_ref, o_ref):
    o_ref[...] = ge_ref[...] @ w_ref[...] + b_ref[...]


def kernel(x_video, x_audio, edge_index_vv, edge_index_aa, edge_index_va, edge_weight_vv, edge_weight_aa, edge_weight_va, batch_video, batch_audio, W_vv, b_vv, W_aa, b_aa, W_src, W_dst, a_src, a_dst, b_va, ln_g_a, ln_b_a, ln_g_v, ln_b_v, att_w, lin_W, lin_b):
    xv, xa = x_video, x_audio
    for i in range(L):
        v_new = _gcn(xv, edge_index_vv, edge_weight_vv, W_vv[i], b_vv[i], N_V)
        a_new = _gcn(xa, edge_index_aa, edge_weight_aa, W_aa[i], b_aa[i], N_A) \
              + _gat(xv, xa, edge_index_va, edge_weight_va, W_src[i], W_dst[i], a_src[i], a_dst[i], b_va[i], N_A)
        xv = jax.nn.relu(v_new)
        xa = jax.nn.relu(a_new)
        xv = _graph_layernorm(xv, batch_video, ln_g_v[i], ln_b_v[i])
        xa = _graph_layernorm(xa, batch_audio, ln_g_a[i], ln_b_a[i])
    gate = xa @ att_w
    gm = jax.ops.segment_max(gate, batch_audio, num_segments=G)
    ge = jnp.exp(gate - gm[batch_audio])
    gs = jax.ops.segment_sum(ge, batch_audio, num_segments=G)
    w = ge / (gs[batch_audio] + 1e-16)
    graph_embed = jax.ops.segment_sum(xa * w[:, None], batch_audio, num_segments=G)
    pred = pl.pallas_call(
        _final_mm_kernel,
        out_shape=jax.ShapeDtypeStruct((G, OUT), jnp.float32),
    )(graph_embed, lin_W, lin_b[None, :])
    return pred


# speedup vs baseline: 3.0766x; 3.0766x over previous
import functools

import jax
import jax.numpy as jnp
from jax import lax
from jax.experimental import pallas as pl
from jax.experimental.pallas import tpu as pltpu
from jax.experimental.pallas import tpu_sc as plsc

_NV = 10000
_NA = 10000
_D = 128
_G = 32
_N2 = _NV + _NA
_BLK = 1000
_NB = 10
_CH = 80
_EVA = 160000
_EVA_PAD = 163840
_NC, _NS = 2, 16
_PREC = lax.Precision.HIGHEST
_F32 = jnp.float32
_I32 = jnp.int32


def _prep_body(x_ref, w1_ref, w2_ref, a_ref,
               h1_ref, h2_ref, e_ref, cmax_ref, mm_ref):
    j = pl.program_id(0)
    m = j // _NB
    x = x_ref[...]
    h1_ref[...] = jnp.dot(x, w1_ref[0], precision=_PREC)
    h2 = jnp.dot(x, w2_ref[0], precision=_PREC)
    h2_ref[...] = h2
    e = jnp.dot(h2, a_ref[0], precision=_PREC)
    e_ref[...] = e
    mx = jnp.max(e)

    @pl.when(j == 0)
    def _():
        mm_ref[0] = _F32(-1e30)
        mm_ref[1] = _F32(-1e30)

    mm_ref[m] = jnp.maximum(mm_ref[m], mx)

    @pl.when(j == 2 * _NB - 1)
    def _():
        cmax_ref[...] = jnp.full((1, _D),
                                 jnp.maximum(mm_ref[0] + mm_ref[1], 0.0), _F32)


def _prep_call(xcat, w1cat, w2cat, acat, interpret=False):
    return pl.pallas_call(
        _prep_body,
        grid=(2 * _NB,),
        in_specs=[
            pl.BlockSpec((_BLK, _D), lambda j: (j, 0)),
            pl.BlockSpec((1, _D, _D), lambda j: (j // _NB, 0, 0)),
            pl.BlockSpec((1, _D, _D), lambda j: (j // _NB, 0, 0)),
            pl.BlockSpec((1, _D, 1), lambda j: (j // _NB, 0, 0)),
        ],
        out_specs=[
            pl.BlockSpec((_BLK, _D), lambda j: (j, 0)),
            pl.BlockSpec((_BLK, _D), lambda j: (j, 0)),
            pl.BlockSpec((_BLK, 1), lambda j: (j, 0)),
            pl.BlockSpec((1, _D), lambda j: (0, 0)),
        ],
        out_shape=[
            jax.ShapeDtypeStruct((_N2, _D), _F32),
            jax.ShapeDtypeStruct((_N2, _D), _F32),
            jax.ShapeDtypeStruct((_N2, 1), _F32),
            jax.ShapeDtypeStruct((1, _D), _F32),
        ],
        scratch_shapes=[pltpu.SMEM((2,), _F32)],
        interpret=interpret,
    )(xcat, w1cat, w2cat, acat)


def _layer_body(accs_ref, accva_ref, bias_ref, g_ref, b_ref, bat_ref,
                w1_ref, w2_ref, a_ref,
                h1_ref, h2_ref, e_ref, cmax_ref,
                xs_ref, st_ref, mm_ref):
    p = pl.program_id(0)
    j = pl.program_id(1)
    m = j // _NB
    mf = jnp.where(j >= _NB, _F32(1.0), _F32(0.0))
    bb = bat_ref[0, 0, 0, :]
    iota = lax.broadcasted_iota(_I32, (1, _G), 1)
    oh = (bb[:, None] == iota).astype(_F32)

    @pl.when(p == 0)
    def _():
        @pl.when(j == 0)
        def _():
            st_ref[...] = jnp.zeros((2, 4, _G), _F32)

        xpre = accs_ref[0] + mf * (accva_ref[0] + accva_ref[1]) + bias_ref[0]
        x1 = jnp.maximum(xpre, 0.0)
        xs_ref[pl.ds(j * _BLK, _BLK), :] = x1
        rs = jnp.sum(x1, axis=1)
        rq = jnp.sum(x1 * x1, axis=1)
        upd = jnp.stack([jnp.dot(rs, oh, precision=_PREC),
                         jnp.dot(rq, oh, precision=_PREC),
                         jnp.sum(oh, axis=0),
                         jnp.zeros((_G,), _F32)])
        st_ref[m] = st_ref[m] + upd

    @pl.when(p == 1)
    def _():
        st = st_ref[m]
        cnt = st[2]
        den = jnp.maximum(cnt * _F32(_D), 1.0)
        mu = st[0] / den
        var = st[1] / den - mu * mu
        rsq = lax.rsqrt(var + 1e-5)
        mub = jnp.dot(oh, mu, precision=_PREC)
        rqb = jnp.dot(oh, rsq, precision=_PREC)
        x1 = xs_ref[pl.ds(j * _BLK, _BLK), :]
        y = (x1 - mub[:, None]) * rqb[:, None] * g_ref[0] + b_ref[0]
        h1_ref[...] = jnp.dot(y, w1_ref[0], precision=_PREC)
        h2 = jnp.dot(y, w2_ref[0], precision=_PREC)
        h2_ref[...] = h2
        e = jnp.dot(h2, a_ref[0], precision=_PREC)
        e_ref[...] = e

        @pl.when(j == 0)
        def _():
            mm_ref[0] = _F32(-1e30)
            mm_ref[1] = _F32(-1e30)

        mm_ref[m] = jnp.maximum(mm_ref[m], jnp.max(e))

        @pl.when(j == 2 * _NB - 1)
        def _():
            cmax_ref[...] = jnp.full(
                (1, _D), jnp.maximum(mm_ref[0] + mm_ref[1], 0.0), _F32)


def _layer_call(accs, accva, biascat, gcat, bcat, batcat, w1cat, w2cat, acat,
                interpret=False):
    return pl.pallas_call(
        _layer_body,
        grid=(2, 2 * _NB),
        in_specs=[
            pl.BlockSpec((1, _BLK, _D), lambda p, j: (j // _NB, j % _NB, 0)),
            pl.BlockSpec((2, _BLK, _D), lambda p, j: (0, j % _NB, 0)),
            pl.BlockSpec((1, 1, _D), lambda p, j: (j // _NB, 0, 0)),
            pl.BlockSpec((1, 1, _D), lambda p, j: (j // _NB, 0, 0)),
            pl.BlockSpec((1, 1, _D), lambda p, j: (j // _NB, 0, 0)),
            pl.BlockSpec((1, 1, 1, _BLK),
                         lambda p, j: (j // _NB, j % _NB, 0, 0)),
            pl.BlockSpec((1, _D, _D), lambda p, j: (j // _NB, 0, 0)),
            pl.BlockSpec((1, _D, _D), lambda p, j: (j // _NB, 0, 0)),
            pl.BlockSpec((1, _D, 1), lambda p, j: (j // _NB, 0, 0)),
        ],
        out_specs=[
            pl.BlockSpec((_BLK, _D), lambda p, j: (j, 0)),
            pl.BlockSpec((_BLK, _D), lambda p, j: (j, 0)),
            pl.BlockSpec((_BLK, 1), lambda p, j: (j, 0)),
            pl.BlockSpec((1, _D), lambda p, j: (0, 0)),
        ],
        out_shape=[
            jax.ShapeDtypeStruct((_N2, _D), _F32),
            jax.ShapeDtypeStruct((_N2, _D), _F32),
            jax.ShapeDtypeStruct((_N2, 1), _F32),
            jax.ShapeDtypeStruct((1, _D), _F32),
        ],
        scratch_shapes=[
            pltpu.VMEM((_N2, _D), _F32),
            pltpu.VMEM((2, 4, _G), _F32),
            pltpu.SMEM((2,), _F32),
        ],
        interpret=interpret,
    )(accs, accva, biascat, gcat, bcat, batcat, w1cat, w2cat, acat)


def _readout_body(accs_ref, accva_ref, bias_ref, g_ref, b_ref, bat_ref,
                  attw_ref, linw_ref, linb_ref,
                  pred_ref, xs_ref, st_ref, u_ref):
    p = pl.program_id(0)
    j = pl.program_id(1)
    bb = bat_ref[0, 0, :]
    iota = lax.broadcasted_iota(_I32, (1, _G), 1)
    oh = (bb[:, None] == iota).astype(_F32)

    @pl.when(p == 0)
    def _():
        @pl.when(j == 0)
        def _():
            st_ref[...] = jnp.zeros((8, _G), _F32)
            st_ref[3, :] = jnp.full((_G,), -1e30, _F32)
            u_ref[...] = jnp.zeros((_G, _D), _F32)

        xpre = (accs_ref[0] + accs_ref[1] + accva_ref[0] + accva_ref[1]
                + bias_ref[...])
        x1 = jnp.maximum(xpre, 0.0)
        xs_ref[pl.ds(j * _BLK, _BLK), :] = x1
        rs = jnp.sum(x1, axis=1)
        rq = jnp.sum(x1 * x1, axis=1)
        st_ref[0, :] = st_ref[0, :] + jnp.dot(rs, oh, precision=_PREC)
        st_ref[1, :] = st_ref[1, :] + jnp.dot(rq, oh, precision=_PREC)
        st_ref[2, :] = st_ref[2, :] + jnp.sum(oh, axis=0)

    @pl.when(p == 1)
    def _():
        cnt = st_ref[2, :]
        den = jnp.maximum(cnt * _F32(_D), 1.0)
        mu = st_ref[0, :] / den
        var = st_ref[1, :] / den - mu * mu
        rsq = lax.rsqrt(var + 1e-5)
        mub = jnp.dot(oh, mu, precision=_PREC)
        rqb = jnp.dot(oh, rsq, precision=_PREC)
        x1 = xs_ref[pl.ds(j * _BLK, _BLK), :]
        y = (x1 - mub[:, None]) * rqb[:, None] * g_ref[...] + b_ref[...]
        xs_ref[pl.ds(j * _BLK, _BLK), :] = y
        gate = jnp.dot(y, attw_ref[...], precision=_PREC)[:, 0]
        gmp = jnp.max(jnp.where(oh > 0, gate[:, None], _F32(-1e30)), axis=0)
        st_ref[3, :] = jnp.maximum(st_ref[3, :], gmp)

    @pl.when(p == 2)
    def _():
        y = xs_ref[pl.ds(j * _BLK, _BLK), :]
        gate = jnp.dot(y, attw_ref[...], precision=_PREC)[:, 0]
        gmb = jnp.dot(oh, st_ref[3, :], precision=_PREC)
        ge = jnp.exp(gate - gmb)
        st_ref[4, :] = st_ref[4, :] + jnp.dot(ge, oh, precision=_PREC)
        u_ref[...] = u_ref[...] + lax.dot_general(
            oh * ge[:, None], y, (((0,), (0,)), ((), ())), precision=_PREC)

        @pl.when(j == _NB - 1)
        def _():
            gs = st_ref[4, :]
            embed = u_ref[...] / (gs[:, None] + 1e-16)
            pred_ref[...] = (jnp.dot(embed, linw_ref[...], precision=_PREC)
                             + linb_ref[...])


def _readout_call(accs, accva, bias_a, gna, bna, bata, attw, linw, linb,
                  interpret=False):
    return pl.pallas_call(
        _readout_body,
        grid=(3, _NB),
        in_specs=[
            pl.BlockSpec((2, _BLK, _D), lambda p, j: (0, j, 0)),
            pl.BlockSpec((2, _BLK, _D), lambda p, j: (0, j, 0)),
            pl.BlockSpec((1, _D), lambda p, j: (0, 0)),
            pl.BlockSpec((1, _D), lambda p, j: (0, 0)),
            pl.BlockSpec((1, _D), lambda p, j: (0, 0)),
            pl.BlockSpec((1, 1, _BLK), lambda p, j: (j, 0, 0)),
            pl.BlockSpec((_D, 1), lambda p, j: (0, 0)),
            pl.BlockSpec((_D, _D), lambda p, j: (0, 0)),
            pl.BlockSpec((1, _D), lambda p, j: (0, 0)),
        ],
        out_specs=pl.BlockSpec((_G, _D), lambda p, j: (0, 0)),
        out_shape=jax.ShapeDtypeStruct((_G, _D), _F32),
        scratch_shapes=[
            pltpu.VMEM((_NA, _D), _F32),
            pltpu.VMEM((8, _G), _F32),
            pltpu.VMEM((_G, _D), _F32),
        ],
        interpret=interpret,
    )(accs, accva, bias_a, gna, bna, bata, attw, linw, linb)


def _scale_rows(rows, wbuf):
    for g in range(_CH // 16):
        w16 = wbuf[pl.ds(g * 16, 16)]
        for jj in range(16):
            wj = w16.at[jnp.full((16,), jj, _I32)].get(
                mode="promise_in_bounds")
            r = g * 16 + jj
            for c in range(_D // 16):
                sl = pl.ds(c * 16, 16)
                rows[r, sl] = rows[r, sl] * wj


def _sc_mesh():
    return plsc.VectorSubcoreMesh(core_axis_name="c", subcore_axis_name="s",
                                  num_cores=_NC, num_subcores=_NS)


def _make_pass1(with_vv, interpret=False):
    n_gcn = 250 if with_vv else 125

    def body(h1_hbm, srcg_hbm, dstg_hbm, ewg_hbm, e1d_hbm,
             srcva_hbm, dstva_hbm, ewva_hbm, cvec_hbm,
             accs_out, ssum_out, ppad_out,
             idx_s, idx_d, idx_d2, wbuf, rows, es, ed, exb, pb, cv,
             zrows, zs, acc, ssum_sh, sem):
        cid = lax.axis_index("c")
        sid = lax.axis_index("s")
        z16 = jnp.zeros((16,), _F32)
        for r in range(24):
            for c in range(_D // 16):
                zrows[r, pl.ds(c * 16, 16)] = z16
        for g in range(_CH // 16):
            zs[pl.ds(g * 16, 16)] = z16

        def zacc(k, carry):
            pltpu.sync_copy(zrows, acc.at[pl.ds(sid * 624 + k * 24, 24)])
            return carry
        lax.fori_loop(0, 26, zacc, 0)

        @pl.when(sid == 0)
        def _():
            pltpu.sync_copy(zrows.at[pl.ds(0, 16)], acc.at[pl.ds(9984, 16)])

        for t in range(_NS):
            @pl.when((cid == 0) & (sid == t))
            def _(t=t):
                for mm in range((125 - t + _NS - 1) // _NS):
                    k = t + _NS * mm
                    pltpu.sync_copy(zs, ssum_sh.at[pl.ds(k * _CH, _CH)])

        @pl.when((cid == 1) & (sid == _NS - 1))
        def _():
            for k in range(48):
                pltpu.sync_copy(zs, ppad_out.at[2000 + k])

        plsc.subcore_barrier()

        if with_vv:
            gcn_base = cid * 4000 + sid * 250
        else:
            gcn_base = (cid * _NS + sid) * 125

        def gcn_chunk(k, carry):
            row = gcn_base + k
            pltpu.sync_copy(srcg_hbm.at[row], idx_s)
            pltpu.sync_copy(ewg_hbm.at[row], wbuf)
            pltpu.async_copy(h1_hbm.at[idx_s], rows, sem).wait()
            _scale_rows(rows, wbuf)
            pltpu.sync_copy(dstg_hbm.at[row], idx_d)
            pltpu.sync_copy(rows, acc.at[idx_d], add=True)
            return carry
        lax.fori_loop(0, n_gcn, gcn_chunk, 0)

        @pl.when(cid == 0)
        def _():
            pltpu.sync_copy(cvec_hbm, cv)
            cvv = cv[...]

            def p1_chunk(k, carry):
                row = sid * 125 + k
                pltpu.sync_copy(srcva_hbm.at[row], idx_s)
                pltpu.sync_copy(dstva_hbm.at[row], idx_d)
                pltpu.sync_copy(ewva_hbm.at[row], wbuf)
                for g in range(_CH // 16):
                    sl = pl.ds(g * 16, 16)
                    idx_d2[sl] = idx_d[sl] + _NV
                pltpu.async_copy(e1d_hbm.at[idx_s], es, sem).wait()
                pltpu.async_copy(e1d_hbm.at[idx_d2], ed, sem).wait()
                for g in range(_CH // 16):
                    sl = pl.ds(g * 16, 16)
                    z = es[sl] + ed[sl]
                    z = jnp.where(z >= 0, z, 0.2 * z)
                    ex = jnp.exp(z - cvv)
                    exb[sl] = ex
                    pb[sl] = ex * wbuf[sl]
                pltpu.sync_copy(exb, ssum_sh.at[idx_d], add=True)
                pltpu.sync_copy(pb, ppad_out.at[row])
                return carry
            lax.fori_loop(0, 125, p1_chunk, 0)

        plsc.subcore_barrier()
        pltpu.sync_copy(acc.at[pl.ds(sid * 624, 624)],
                        accs_out.at[cid, pl.ds(sid * 624, 624)])

        @pl.when(sid == 0)
        def _():
            pltpu.sync_copy(acc.at[pl.ds(9984, 16)],
                            accs_out.at[cid, pl.ds(9984, 16)])

        @pl.when((cid == 0) & (sid == 0))
        def _():
            pltpu.sync_copy(ssum_sh, ssum_out)

    return pl.kernel(
        body,
        out_type=[
            jax.ShapeDtypeStruct((2, _NV, _D), _F32),
            jax.ShapeDtypeStruct((_NA,), _F32),
            jax.ShapeDtypeStruct((_EVA_PAD // _CH, _CH), _F32),
        ],
        mesh=_sc_mesh(),
        scratch_types=[
            pltpu.VMEM((_CH,), _I32),
            pltpu.VMEM((_CH,), _I32),
            pltpu.VMEM((_CH,), _I32),
            pltpu.VMEM((_CH,), _F32),
            pltpu.VMEM((_CH, _D), _F32),
            pltpu.VMEM((_CH,), _F32),
            pltpu.VMEM((_CH,), _F32),
            pltpu.VMEM((_CH,), _F32),
            pltpu.VMEM((_CH,), _F32),
            pltpu.VMEM((16,), _F32),
            pltpu.VMEM((24, _D), _F32),
            pltpu.VMEM((_CH,), _F32),
            pltpu.VMEM_SHARED((_NV, _D), _F32),
            pltpu.VMEM_SHARED((_NA,), _F32),
            pltpu.SemaphoreType.DMA,
        ],
        interpret=interpret,
    )


def _make_pass2(interpret=False):
    def body(h2_hbm, srcp_hbm, dstp_hbm, pp_hbm, ssum_hbm,
             accva_out,
             idx_s, idx_d, pch, sden, wbuf, rows, zrows, acc, sem):
        cid = lax.axis_index("c")
        sid = lax.axis_index("s")
        wid = cid * _NS + sid
        z16 = jnp.zeros((16,), _F32)
        for r in range(24):
            for c in range(_D // 16):
                zrows[r, pl.ds(c * 16, 16)] = z16

        def zacc(k, carry):
            pltpu.sync_copy(zrows, acc.at[pl.ds(sid * 624 + k * 24, 24)])
            return carry
        lax.fori_loop(0, 26, zacc, 0)

        @pl.when(sid == 0)
        def _():
            pltpu.sync_copy(zrows.at[pl.ds(0, 16)], acc.at[pl.ds(9984, 16)])

        plsc.subcore_barrier()

        def chunk(k, carry):
            row = wid * 64 + k
            pltpu.sync_copy(srcp_hbm.at[row], idx_s)
            pltpu.sync_copy(dstp_hbm.at[row], idx_d)
            pltpu.sync_copy(pp_hbm.at[row], pch)
            pltpu.async_copy(ssum_hbm.at[idx_d], sden, sem).wait()
            pltpu.async_copy(h2_hbm.at[idx_s], rows, sem).wait()
            for g in range(_CH // 16):
                sl = pl.ds(g * 16, 16)
                wbuf[sl] = pch[sl] / (sden[sl] + 1e-16)
            _scale_rows(rows, wbuf)
            pltpu.sync_copy(rows, acc.at[idx_d], add=True)
            return carry
        lax.fori_loop(0, 64, chunk, 0)

        plsc.subcore_barrier()
        pltpu.sync_copy(acc.at[pl.ds(sid * 624, 624)],
                        accva_out.at[cid, pl.ds(sid * 624, 624)])

        @pl.when(sid == 0)
        def _():
            pltpu.sync_copy(acc.at[pl.ds(9984, 16)],
                            accva_out.at[cid, pl.ds(9984, 16)])

    return pl.kernel(
        body,
        out_type=jax.ShapeDtypeStruct((2, _NA, _D), _F32),
        mesh=_sc_mesh(),
        scratch_types=[
            pltpu.VMEM((_CH,), _I32),
            pltpu.VMEM((_CH,), _I32),
            pltpu.VMEM((_CH,), _F32),
            pltpu.VMEM((_CH,), _F32),
            pltpu.VMEM((_CH,), _F32),
            pltpu.VMEM((_CH, _D), _F32),
            pltpu.VMEM((24, _D), _F32),
            pltpu.VMEM_SHARED((_NA, _D), _F32),
            pltpu.SemaphoreType.DMA,
        ],
        interpret=interpret,
    )


def kernel(x_video, x_audio, edge_index_vv, edge_index_aa, edge_index_va,
           edge_weight_vv, edge_weight_aa, edge_weight_va,
           batch_video, batch_audio,
           W_vv, b_vv, W_aa, b_aa, W_src, W_dst, a_src, a_dst, b_va,
           ln_g_a, ln_b_a, ln_g_v, ln_b_v, att_w, lin_W, lin_b):
    xv = x_video.astype(_F32)
    xa = x_audio.astype(_F32)
    src_vv = edge_index_vv[0].astype(_I32)
    dst_vv = edge_index_vv[1].astype(_I32)
    src_aa = edge_index_aa[0].astype(_I32)
    dst_aa = edge_index_aa[1].astype(_I32)
    src_va = edge_index_va[0].astype(_I32)
    dst_va = edge_index_va[1].astype(_I32)

    srcg = jnp.concatenate([src_vv, src_aa + _NV]).reshape(-1, _CH)
    dstg = jnp.concatenate([dst_vv, dst_aa]).reshape(-1, _CH)
    ewg = jnp.concatenate([edge_weight_vv, edge_weight_aa]).reshape(-1, _CH)
    srcg_aa = (src_aa + _NV).reshape(-1, _CH)
    dstg_aa = dst_aa.reshape(-1, _CH)
    ewg_aa = edge_weight_aa.reshape(-1, _CH)

    pad = _EVA_PAD - _EVA
    srcpad = jnp.pad(src_va, (0, pad)).reshape(-1, _CH)
    dstpad = jnp.pad(dst_va, (0, pad)).reshape(-1, _CH)
    srcva2 = src_va.reshape(-1, _CH)
    dstva2 = dst_va.reshape(-1, _CH)
    ewva2 = edge_weight_va.reshape(-1, _CH)

    xcat = jnp.concatenate([xv, xa])
    batv3 = batch_video.astype(_I32).reshape(_NB, 1, _BLK)
    bata3 = batch_audio.astype(_I32).reshape(_NB, 1, _BLK)
    batcat = jnp.stack([batv3, bata3])

    def layer_weights(i):
        w1 = jnp.stack([W_vv[i], W_aa[i]])
        w2 = jnp.stack([W_src[i], W_dst[i]])
        a = jnp.stack([a_src[i][:, None], a_dst[i][:, None]])
        return w1, w2, a

    p1_full = _make_pass1(True)
    p1_last = _make_pass1(False)
    p2 = _make_pass2()

    w1c, w2c, ac = layer_weights(0)
    h1, h2, ecat, cmax = _prep_call(xcat, w1c, w2c, ac)
    pred = None
    for i in range(4):
        e1d = ecat.reshape(_N2)
        cvec = cmax[0, :16]
        if i < 3:
            accs, ssum, ppad = p1_full(h1, srcg, dstg, ewg, e1d,
                                       srcva2, dstva2, ewva2, cvec)
        else:
            accs, ssum, ppad = p1_last(h1, srcg_aa, dstg_aa, ewg_aa, e1d,
                                       srcva2, dstva2, ewva2, cvec)
        accva = p2(h2, srcpad, dstpad, ppad, ssum)
        if i < 3:
            biascat = jnp.stack([b_vv[i][None, :],
                                 (b_aa[i] + b_va[i])[None, :]])
            gcat = jnp.stack([ln_g_v[i][None, :], ln_g_a[i][None, :]])
            bcat = jnp.stack([ln_b_v[i][None, :], ln_b_a[i][None, :]])
            w1c, w2c, ac = layer_weights(i + 1)
            h1, h2, ecat, cmax = _layer_call(accs, accva, biascat, gcat,
                                             bcat, batcat, w1c, w2c, ac)
        else:
            bias_a = (b_aa[i] + b_va[i])[None, :]
            pred = _readout_call(accs, accva, bias_a, ln_g_a[i][None, :],
                                 ln_b_a[i][None, :], bata3,
                                 att_w[:, None], lin_W, lin_b[None, :])
    return pred


# speedup vs baseline: 4.9196x; 1.5991x over previous
import functools

import jax
import jax.numpy as jnp
from jax import lax
from jax.experimental import pallas as pl
from jax.experimental.pallas import tpu as pltpu
from jax.experimental.pallas import tpu_sc as plsc

_NV = 10000
_NA = 10000
_D = 128
_G = 32
_N2 = _NV + _NA
_BLK = 1000
_NB = 10
_CH = 80
_EVA = 160000
_EVA_PAD = 163840
_NC, _NS = 2, 16
_PREC = None
_F32 = jnp.float32
_I32 = jnp.int32


def _prep_body(x_ref, w1_ref, w2_ref, a_ref,
               h1_ref, h2_ref, e_ref, cmax_ref, mm_ref):
    j = pl.program_id(0)
    m = j // _NB
    x = x_ref[...]
    h1_ref[...] = jnp.dot(x, w1_ref[0], precision=_PREC)
    h2 = jnp.dot(x, w2_ref[0], precision=_PREC)
    h2_ref[...] = h2
    e = jnp.dot(h2, a_ref[0], precision=_PREC)
    e_ref[...] = e
    mx = jnp.max(e)

    @pl.when(j == 0)
    def _():
        mm_ref[0] = _F32(-1e30)
        mm_ref[1] = _F32(-1e30)

    mm_ref[m] = jnp.maximum(mm_ref[m], mx)

    @pl.when(j == 2 * _NB - 1)
    def _():
        cmax_ref[...] = jnp.full((1, _D),
                                 jnp.maximum(mm_ref[0] + mm_ref[1], 0.0), _F32)


def _prep_call(xcat, w1cat, w2cat, acat, interpret=False):
    return pl.pallas_call(
        _prep_body,
        grid=(2 * _NB,),
        in_specs=[
            pl.BlockSpec((_BLK, _D), lambda j: (j, 0)),
            pl.BlockSpec((1, _D, _D), lambda j: (j // _NB, 0, 0)),
            pl.BlockSpec((1, _D, _D), lambda j: (j // _NB, 0, 0)),
            pl.BlockSpec((1, _D, 1), lambda j: (j // _NB, 0, 0)),
        ],
        out_specs=[
            pl.BlockSpec((_BLK, _D), lambda j: (j, 0)),
            pl.BlockSpec((_BLK, _D), lambda j: (j, 0)),
            pl.BlockSpec((_BLK, 1), lambda j: (j, 0)),
            pl.BlockSpec((1, _D), lambda j: (0, 0)),
        ],
        out_shape=[
            jax.ShapeDtypeStruct((_N2, _D), _F32),
            jax.ShapeDtypeStruct((_N2, _D), _F32),
            jax.ShapeDtypeStruct((_N2, 1), _F32),
            jax.ShapeDtypeStruct((1, _D), _F32),
        ],
        scratch_shapes=[pltpu.SMEM((2,), _F32)],
        interpret=interpret,
    )(xcat, w1cat, w2cat, acat)


def _layer_body(accs_ref, accva_ref, bias_ref, g_ref, b_ref, bat_ref,
                w1_ref, w2_ref, a_ref,
                h1_ref, h2_ref, e_ref, cmax_ref,
                xs_ref, st_ref, mm_ref):
    p = pl.program_id(0)
    j = pl.program_id(1)
    m = j // _NB
    mf = jnp.where(j >= _NB, _F32(1.0), _F32(0.0))
    bb = bat_ref[0, 0, 0, :]
    iota = lax.broadcasted_iota(_I32, (1, _G), 1)
    oh = (bb[:, None] == iota).astype(_F32)

    @pl.when(p == 0)
    def _():
        @pl.when(j == 0)
        def _():
            st_ref[...] = jnp.zeros((2, 4, _G), _F32)

        xpre = accs_ref[0] + mf * (accva_ref[0] + accva_ref[1]) + bias_ref[0]
        x1 = jnp.maximum(xpre, 0.0)
        xs_ref[pl.ds(j * _BLK, _BLK), :] = x1
        rs = jnp.sum(x1, axis=1)
        rq = jnp.sum(x1 * x1, axis=1)
        upd = jnp.stack([jnp.dot(rs, oh, precision=_PREC),
                         jnp.dot(rq, oh, precision=_PREC),
                         jnp.sum(oh, axis=0),
                         jnp.zeros((_G,), _F32)])
        st_ref[m] = st_ref[m] + upd

    @pl.when(p == 1)
    def _():
        st = st_ref[m]
        cnt = st[2]
        den = jnp.maximum(cnt * _F32(_D), 1.0)
        mu = st[0] / den
        var = st[1] / den - mu * mu
        rsq = lax.rsqrt(var + 1e-5)
        mub = jnp.dot(oh, mu, precision=_PREC)
        rqb = jnp.dot(oh, rsq, precision=_PREC)
        x1 = xs_ref[pl.ds(j * _BLK, _BLK), :]
        y = (x1 - mub[:, None]) * rqb[:, None] * g_ref[0] + b_ref[0]
        h1_ref[...] = jnp.dot(y, w1_ref[0], precision=_PREC)
        h2 = jnp.dot(y, w2_ref[0], precision=_PREC)
        h2_ref[...] = h2
        e = jnp.dot(h2, a_ref[0], precision=_PREC)
        e_ref[...] = e

        @pl.when(j == 0)
        def _():
            mm_ref[0] = _F32(-1e30)
            mm_ref[1] = _F32(-1e30)

        mm_ref[m] = jnp.maximum(mm_ref[m], jnp.max(e))

        @pl.when(j == 2 * _NB - 1)
        def _():
            cmax_ref[...] = jnp.full(
                (1, _D), jnp.maximum(mm_ref[0] + mm_ref[1], 0.0), _F32)


def _layer_call(accs, accva, biascat, gcat, bcat, batcat, w1cat, w2cat, acat,
                interpret=False):
    return pl.pallas_call(
        _layer_body,
        grid=(2, 2 * _NB),
        in_specs=[
            pl.BlockSpec((1, _BLK, _D), lambda p, j: (j // _NB, j % _NB, 0)),
            pl.BlockSpec((2, _BLK, _D), lambda p, j: (0, j % _NB, 0)),
            pl.BlockSpec((1, 1, _D), lambda p, j: (j // _NB, 0, 0)),
            pl.BlockSpec((1, 1, _D), lambda p, j: (j // _NB, 0, 0)),
            pl.BlockSpec((1, 1, _D), lambda p, j: (j // _NB, 0, 0)),
            pl.BlockSpec((1, 1, 1, _BLK),
                         lambda p, j: (j // _NB, j % _NB, 0, 0)),
            pl.BlockSpec((1, _D, _D), lambda p, j: (j // _NB, 0, 0)),
            pl.BlockSpec((1, _D, _D), lambda p, j: (j // _NB, 0, 0)),
            pl.BlockSpec((1, _D, 1), lambda p, j: (j // _NB, 0, 0)),
        ],
        out_specs=[
            pl.BlockSpec((_BLK, _D), lambda p, j: (j, 0)),
            pl.BlockSpec((_BLK, _D), lambda p, j: (j, 0)),
            pl.BlockSpec((_BLK, 1), lambda p, j: (j, 0)),
            pl.BlockSpec((1, _D), lambda p, j: (0, 0)),
        ],
        out_shape=[
            jax.ShapeDtypeStruct((_N2, _D), _F32),
            jax.ShapeDtypeStruct((_N2, _D), _F32),
            jax.ShapeDtypeStruct((_N2, 1), _F32),
            jax.ShapeDtypeStruct((1, _D), _F32),
        ],
        scratch_shapes=[
            pltpu.VMEM((_N2, _D), _F32),
            pltpu.VMEM((2, 4, _G), _F32),
            pltpu.SMEM((2,), _F32),
        ],
        interpret=interpret,
    )(accs, accva, biascat, gcat, bcat, batcat, w1cat, w2cat, acat)


def _readout_body(accs_ref, accva_ref, bias_ref, g_ref, b_ref, bat_ref,
                  attw_ref, linw_ref, linb_ref,
                  pred_ref, xs_ref, st_ref, u_ref):
    p = pl.program_id(0)
    j = pl.program_id(1)
    bb = bat_ref[0, 0, :]
    iota = lax.broadcasted_iota(_I32, (1, _G), 1)
    oh = (bb[:, None] == iota).astype(_F32)

    @pl.when(p == 0)
    def _():
        @pl.when(j == 0)
        def _():
            st_ref[...] = jnp.zeros((8, _G), _F32)
            st_ref[3, :] = jnp.full((_G,), -1e30, _F32)
            u_ref[...] = jnp.zeros((_G, _D), _F32)

        xpre = (accs_ref[0] + accs_ref[1] + accva_ref[0] + accva_ref[1]
                + bias_ref[...])
        x1 = jnp.maximum(xpre, 0.0)
        xs_ref[pl.ds(j * _BLK, _BLK), :] = x1
        rs = jnp.sum(x1, axis=1)
        rq = jnp.sum(x1 * x1, axis=1)
        st_ref[0, :] = st_ref[0, :] + jnp.dot(rs, oh, precision=_PREC)
        st_ref[1, :] = st_ref[1, :] + jnp.dot(rq, oh, precision=_PREC)
        st_ref[2, :] = st_ref[2, :] + jnp.sum(oh, axis=0)

    @pl.when(p == 1)
    def _():
        cnt = st_ref[2, :]
        den = jnp.maximum(cnt * _F32(_D), 1.0)
        mu = st_ref[0, :] / den
        var = st_ref[1, :] / den - mu * mu
        rsq = lax.rsqrt(var + 1e-5)
        mub = jnp.dot(oh, mu, precision=_PREC)
        rqb = jnp.dot(oh, rsq, precision=_PREC)
        x1 = xs_ref[pl.ds(j * _BLK, _BLK), :]
        y = (x1 - mub[:, None]) * rqb[:, None] * g_ref[...] + b_ref[...]
        xs_ref[pl.ds(j * _BLK, _BLK), :] = y
        gate = jnp.dot(y, attw_ref[...], precision=_PREC)[:, 0]
        gmp = jnp.max(jnp.where(oh > 0, gate[:, None], _F32(-1e30)), axis=0)
        st_ref[3, :] = jnp.maximum(st_ref[3, :], gmp)

    @pl.when(p == 2)
    def _():
        y = xs_ref[pl.ds(j * _BLK, _BLK), :]
        gate = jnp.dot(y, attw_ref[...], precision=_PREC)[:, 0]
        gmb = jnp.dot(oh, st_ref[3, :], precision=_PREC)
        ge = jnp.exp(gate - gmb)
        st_ref[4, :] = st_ref[4, :] + jnp.dot(ge, oh, precision=_PREC)
        u_ref[...] = u_ref[...] + lax.dot_general(
            oh * ge[:, None], y, (((0,), (0,)), ((), ())), precision=_PREC)

        @pl.when(j == _NB - 1)
        def _():
            gs = st_ref[4, :]
            embed = u_ref[...] / (gs[:, None] + 1e-16)
            pred_ref[...] = (jnp.dot(embed, linw_ref[...], precision=_PREC)
                             + linb_ref[...])


def _readout_call(accs, accva, bias_a, gna, bna, bata, attw, linw, linb,
                  interpret=False):
    return pl.pallas_call(
        _readout_body,
        grid=(3, _NB),
        in_specs=[
            pl.BlockSpec((2, _BLK, _D), lambda p, j: (0, j, 0)),
            pl.BlockSpec((2, _BLK, _D), lambda p, j: (0, j, 0)),
            pl.BlockSpec((1, _D), lambda p, j: (0, 0)),
            pl.BlockSpec((1, _D), lambda p, j: (0, 0)),
            pl.BlockSpec((1, _D), lambda p, j: (0, 0)),
            pl.BlockSpec((1, 1, _BLK), lambda p, j: (j, 0, 0)),
            pl.BlockSpec((_D, 1), lambda p, j: (0, 0)),
            pl.BlockSpec((_D, _D), lambda p, j: (0, 0)),
            pl.BlockSpec((1, _D), lambda p, j: (0, 0)),
        ],
        out_specs=pl.BlockSpec((_G, _D), lambda p, j: (0, 0)),
        out_shape=jax.ShapeDtypeStruct((_G, _D), _F32),
        scratch_shapes=[
            pltpu.VMEM((_NA, _D), _F32),
            pltpu.VMEM((8, _G), _F32),
            pltpu.VMEM((_G, _D), _F32),
        ],
        interpret=interpret,
    )(accs, accva, bias_a, gna, bna, bata, attw, linw, linb)


def _scale_rows(rows, wbuf):
    for g in range(_CH // 16):
        w16 = wbuf[pl.ds(g * 16, 16)]
        for jj in range(16):
            wj = w16.at[jnp.full((16,), jj, _I32)].get(
                mode="promise_in_bounds")
            r = g * 16 + jj
            for c in range(_D // 16):
                sl = pl.ds(c * 16, 16)
                rows[r, sl] = rows[r, sl] * wj


def _sc_mesh():
    return plsc.VectorSubcoreMesh(core_axis_name="c", subcore_axis_name="s",
                                  num_cores=_NC, num_subcores=_NS)


def _make_pass1(with_vv, interpret=False):
    def body(h1_hbm, srcg_hbm, dstg_hbm, ewg_hbm, e1d_hbm,
             srcva_hbm, dstva_hbm, ewva_hbm, cvec_hbm,
             accs_out, ssum_out, ppad_out,
             is0, id0, id20, w0, r0, es0, ed0, ex0, pb0, cv,
             zrows, zs, acc, ssum_sh,
             s_sm, s_g, s_g2, s_s):
        cid = lax.axis_index("c")
        sid = lax.axis_index("s")
        z16 = jnp.zeros((16,), _F32)
        for r in range(24):
            for c in range(_D // 16):
                zrows[r, pl.ds(c * 16, 16)] = z16
        for g in range(_CH // 16):
            zs[pl.ds(g * 16, 16)] = z16

        def zacc(k, carry):
            pltpu.sync_copy(zrows, acc.at[pl.ds(sid * 624 + k * 24, 24)])
            return carry
        lax.fori_loop(0, 26, zacc, 0)

        @pl.when(sid == 0)
        def _():
            pltpu.sync_copy(zrows.at[pl.ds(0, 16)], acc.at[pl.ds(9984, 16)])

        for t in range(_NS):
            @pl.when(sid == t)
            def _(t=t):
                for mm in range((126 - t + _NS - 1) // _NS):
                    k = t + _NS * mm
                    pltpu.sync_copy(zs, ssum_sh.at[pl.ds(k * _CH, _CH)])

        plsc.subcore_barrier()

        if with_vv:
            gcn_base = cid * 4000 + sid * 250
            n_gcn = 250
        else:
            gcn_base = (cid * _NS + sid) * 126
            n_gcn = 126

        def gcn_chunk(k, carry):
            row = gcn_base + k
            a = pltpu.async_copy(srcg_hbm.at[row], is0, s_sm)
            b = pltpu.async_copy(ewg_hbm.at[row], w0, s_sm)
            c = pltpu.async_copy(dstg_hbm.at[row], id0, s_sm)
            a.wait()
            b.wait()
            c.wait()
            pltpu.async_copy(h1_hbm.at[is0], r0, s_g).wait()
            _scale_rows(r0, w0)
            pltpu.async_copy(r0, acc.at[id0], s_s, add=True).wait()
            return carry
        lax.fori_loop(0, n_gcn, gcn_chunk, 0)

        pltpu.sync_copy(cvec_hbm, cv)
        cvv = cv[...]
        rbase = (cid * _NS + sid) * 64

        def p1_chunk(k, carry):
            row = rbase + k
            a = pltpu.async_copy(srcva_hbm.at[row], is0, s_sm)
            b = pltpu.async_copy(dstva_hbm.at[row], id0, s_sm)
            c = pltpu.async_copy(ewva_hbm.at[row], w0, s_sm)
            a.wait()
            b.wait()
            c.wait()
            for g2 in range(_CH // 16):
                sl = pl.ds(g2 * 16, 16)
                id20[sl] = id0[sl] + _NV
            g = pltpu.async_copy(e1d_hbm.at[is0], es0, s_g)
            h = pltpu.async_copy(e1d_hbm.at[id20], ed0, s_g2)
            g.wait()
            h.wait()
            for g2 in range(_CH // 16):
                sl = pl.ds(g2 * 16, 16)
                z = es0[sl] + ed0[sl]
                z = jnp.where(z >= 0, z, 0.2 * z)
                ex = jnp.exp(z - cvv)
                ex0[sl] = ex
                pb0[sl] = ex * w0[sl]
            pltpu.async_copy(ex0, ssum_sh.at[id0], s_s, add=True).wait()
            pltpu.async_copy(pb0, ppad_out.at[row], s_s).wait()
            return carry
        lax.fori_loop(0, 64, p1_chunk, 0)

        plsc.subcore_barrier()
        pltpu.sync_copy(acc.at[pl.ds(sid * 624, 624)],
                        accs_out.at[cid, pl.ds(sid * 624, 624)])

        @pl.when(sid == 0)
        def _():
            pltpu.sync_copy(acc.at[pl.ds(9984, 16)],
                            accs_out.at[cid, pl.ds(9984, 16)])

        @pl.when(sid == 0)
        def _():
            pltpu.sync_copy(ssum_sh, ssum_out.at[cid])

    return pl.kernel(
        body,
        out_type=[
            jax.ShapeDtypeStruct((2, _NV, _D), _F32),
            jax.ShapeDtypeStruct((2, _NA + _CH), _F32),
            jax.ShapeDtypeStruct((_EVA_PAD // _CH, _CH), _F32),
        ],
        mesh=_sc_mesh(),
        scratch_types=(
            [pltpu.VMEM((_CH,), _I32)] * 3
            + [pltpu.VMEM((_CH,), _F32),
               pltpu.VMEM((_CH, _D), _F32)]
            + [pltpu.VMEM((_CH,), _F32)] * 4
            + [pltpu.VMEM((16,), _F32),
               pltpu.VMEM((24, _D), _F32),
               pltpu.VMEM((_CH,), _F32),
               pltpu.VMEM_SHARED((_NV, _D), _F32),
               pltpu.VMEM_SHARED((_NA + _CH,), _F32)]
            + [pltpu.SemaphoreType.DMA] * 4
        ),
        interpret=interpret,
    )


def _make_pass2(interpret=False):
    def body(h2_hbm, srcp_hbm, dstp_hbm, pp_hbm, ss0_hbm, ss1_hbm,
             accva_out,
             is0, id0, pc0, sa0, sb0, w0, r0, zrows, acc,
             s_sm, s_g, s_g2, s_s):
        cid = lax.axis_index("c")
        sid = lax.axis_index("s")
        wid = cid * _NS + sid
        z16 = jnp.zeros((16,), _F32)
        for r in range(24):
            for c in range(_D // 16):
                zrows[r, pl.ds(c * 16, 16)] = z16

        def zacc(k, carry):
            pltpu.sync_copy(zrows, acc.at[pl.ds(sid * 624 + k * 24, 24)])
            return carry
        lax.fori_loop(0, 26, zacc, 0)

        @pl.when(sid == 0)
        def _():
            pltpu.sync_copy(zrows.at[pl.ds(0, 16)], acc.at[pl.ds(9984, 16)])

        plsc.subcore_barrier()

        def chunk(k, carry):
            row = wid * 64 + k
            a = pltpu.async_copy(srcp_hbm.at[row], is0, s_sm)
            b = pltpu.async_copy(dstp_hbm.at[row], id0, s_sm)
            c = pltpu.async_copy(pp_hbm.at[row], pc0, s_sm)
            a.wait()
            b.wait()
            c.wait()
            g = pltpu.async_copy(h2_hbm.at[is0], r0, s_g)
            u = pltpu.async_copy(ss0_hbm.at[id0], sa0, s_g2)
            v = pltpu.async_copy(ss1_hbm.at[id0], sb0, s_g2)
            g.wait()
            u.wait()
            v.wait()
            for g2 in range(_CH // 16):
                sl = pl.ds(g2 * 16, 16)
                w0[sl] = pc0[sl] / (sa0[sl] + sb0[sl] + 1e-16)
            _scale_rows(r0, w0)
            pltpu.async_copy(r0, acc.at[id0], s_s, add=True).wait()
            return carry
        lax.fori_loop(0, 64, chunk, 0)

        plsc.subcore_barrier()
        pltpu.sync_copy(acc.at[pl.ds(sid * 624, 624)],
                        accva_out.at[cid, pl.ds(sid * 624, 624)])

        @pl.when(sid == 0)
        def _():
            pltpu.sync_copy(acc.at[pl.ds(9984, 16)],
                            accva_out.at[cid, pl.ds(9984, 16)])

    return pl.kernel(
        body,
        out_type=jax.ShapeDtypeStruct((2, _NA, _D), _F32),
        mesh=_sc_mesh(),
        scratch_types=(
            [pltpu.VMEM((_CH,), _I32)] * 2
            + [pltpu.VMEM((_CH,), _F32)] * 4
            + [pltpu.VMEM((_CH, _D), _F32),
               pltpu.VMEM((24, _D), _F32),
               pltpu.VMEM_SHARED((_NA, _D), _F32)]
            + [pltpu.SemaphoreType.DMA] * 4
        ),
        interpret=interpret,
    )


def kernel(x_video, x_audio, edge_index_vv, edge_index_aa, edge_index_va,
           edge_weight_vv, edge_weight_aa, edge_weight_va,
           batch_video, batch_audio,
           W_vv, b_vv, W_aa, b_aa, W_src, W_dst, a_src, a_dst, b_va,
           ln_g_a, ln_b_a, ln_g_v, ln_b_v, att_w, lin_W, lin_b):
    xv = x_video.astype(_F32)
    xa = x_audio.astype(_F32)
    src_vv = edge_index_vv[0].astype(_I32)
    dst_vv = edge_index_vv[1].astype(_I32)
    src_aa = edge_index_aa[0].astype(_I32)
    dst_aa = edge_index_aa[1].astype(_I32)
    src_va = edge_index_va[0].astype(_I32)
    dst_va = edge_index_va[1].astype(_I32)

    srcg = jnp.concatenate([src_vv, src_aa + _NV]).reshape(-1, _CH)
    dstg = jnp.concatenate([dst_vv, dst_aa]).reshape(-1, _CH)
    ewg = jnp.concatenate([edge_weight_vv, edge_weight_aa]).reshape(-1, _CH)
    pad_aa = 4032 * _CH - src_aa.shape[0]
    srcg_aa = jnp.pad(src_aa + _NV, (0, pad_aa)).reshape(-1, _CH)
    dstg_aa = jnp.pad(dst_aa, (0, pad_aa)).reshape(-1, _CH)
    ewg_aa = jnp.pad(edge_weight_aa, (0, pad_aa)).reshape(-1, _CH)

    pad = _EVA_PAD - _EVA
    srcpad = jnp.pad(src_va, (0, pad)).reshape(-1, _CH)
    dstpad = jnp.pad(dst_va, (0, pad)).reshape(-1, _CH)
    srcva2 = srcpad
    dstva2 = jnp.pad(dst_va, (0, pad), constant_values=_NA).reshape(-1, _CH)
    ewva2 = jnp.pad(edge_weight_va, (0, pad)).reshape(-1, _CH)

    xcat = jnp.concatenate([xv, xa])
    batv3 = batch_video.astype(_I32).reshape(_NB, 1, _BLK)
    bata3 = batch_audio.astype(_I32).reshape(_NB, 1, _BLK)
    batcat = jnp.stack([batv3, bata3])

    def layer_weights(i):
        w1 = jnp.stack([W_vv[i], W_aa[i]])
        w2 = jnp.stack([W_src[i], W_dst[i]])
        a = jnp.stack([a_src[i][:, None], a_dst[i][:, None]])
        return w1, w2, a

    p1_full = _make_pass1(True)
    p1_last = _make_pass1(False)
    p2 = _make_pass2()

    w1c, w2c, ac = layer_weights(0)
    h1, h2, ecat, cmax = _prep_call(xcat, w1c, w2c, ac)
    pred = None
    for i in range(4):
        e1d = jnp.pad(ecat.reshape(_N2), (0, 16))
        cvec = cmax[0, :16]
        if i < 3:
            accs, ssum, ppad = p1_full(h1, srcg, dstg, ewg, e1d,
                                       srcva2, dstva2, ewva2, cvec)
        else:
            accs, ssum, ppad = p1_last(h1, srcg_aa, dstg_aa, ewg_aa, e1d,
                                       srcva2, dstva2, ewva2, cvec)
        accva = p2(h2, srcpad, dstpad, ppad, ssum[0], ssum[1])
        if i < 3:
            biascat = jnp.stack([b_vv[i][None, :],
                                 (b_aa[i] + b_va[i])[None, :]])
            gcat = jnp.stack([ln_g_v[i][None, :], ln_g_a[i][None, :]])
            bcat = jnp.stack([ln_b_v[i][None, :], ln_b_a[i][None, :]])
            w1c, w2c, ac = layer_weights(i + 1)
            h1, h2, ecat, cmax = _layer_call(accs, accva, biascat, gcat,
                                             bcat, batcat, w1c, w2c, ac)
        else:
            bias_a = (b_aa[i] + b_va[i])[None, :]
            pred = _readout_call(accs, accva, bias_a, ln_g_a[i][None, :],
                                 ln_b_a[i][None, :], bata3,
                                 att_w[:, None], lin_W, lin_b[None, :])
    return pred
